# baseline (speedup 1.0000x reference)
import jax
import jax.numpy as jnp
from jax.experimental import pallas as pl

B = 1024
IN = 784
H = 400
D = 256
K = 512


def _mmb(a, b_t):
    return jax.lax.dot_general(a.astype(jnp.bfloat16),
                               b_t.astype(jnp.bfloat16),
                               (((1,), (1,)), ((), ())),
                               preferred_element_type=jnp.float32)


def _mm_hi(a, b_t):
    return jax.lax.dot_general(a, b_t, (((1,), (1,)), ((), ())),
                               preferred_element_type=jnp.float32,
                               precision=jax.lax.Precision.HIGHEST)


def _vqvae_kernel(x_ref, fc1_w_ref, fc1_b_ref, fc2_w_ref, fc2_b_ref,
                  fc3_w_ref, fc3_b_ref, fc4_w_ref, fc4_b_ref, emb_ref,
                  xr_ref, rloss_ref, eloss_ref):
    f32 = jnp.float32
    x = x_ref[...]
    h1 = jnp.maximum(_mmb(x, fc1_w_ref[...]) + fc1_b_ref[...], 0.0)
    z_e = _mmb(h1, fc2_w_ref[...]) + fc2_b_ref[...]
    emb = emb_ref[...]
    g = _mm_hi(z_e, emb)
    emb_sq = _mm_hi(jnp.ones((1, D), f32), emb * emb)
    score = emb_sq - 2.0 * g
    m = jnp.min(score, axis=1, keepdims=True)
    lane = jax.lax.broadcasted_iota(jnp.int32, (B, K), 1)
    idx = jnp.min(jnp.where(score == m, lane, K), axis=1, keepdims=True)
    onehot = (lane == idx).astype(jnp.bfloat16)
    z_q = jax.lax.dot_general(onehot, emb.astype(jnp.bfloat16),
                              (((1,), (0,)), ((), ())),
                              preferred_element_type=f32)
    h3 = jnp.maximum(_mmb(z_q, fc3_w_ref[...]) + fc3_b_ref[...], 0.0)
    logits = _mmb(h3, fc4_w_ref[...]) + fc4_b_ref[...]
    en = jnp.exp(-logits)
    x_reconst = 1.0 / (1.0 + en)
    xr_ref[...] = x_reconst
    c = jnp.log(1.0 + en)
    logp = jnp.maximum(-c, -100.0)
    log1mp = jnp.maximum(-logits - c, -100.0)
    rloss_ref[0] = -jnp.sum(x * logp + (1.0 - x) * log1mp) / (B * IN)
    dz = z_e - z_q
    eloss_ref[0] = jnp.sum(dz * dz) / B


def kernel(x, fc1_w, fc1_b, fc2_w, fc2_b, fc3_w, fc3_b, fc4_w, fc4_b, emb):
    from jax.experimental.pallas import tpu as pltpu
    out = pl.pallas_call(
        _vqvae_kernel,
        out_shape=(
            jax.ShapeDtypeStruct((B, IN), jnp.float32),
            jax.ShapeDtypeStruct((1,), jnp.float32),
            jax.ShapeDtypeStruct((1,), jnp.float32),
        ),
        out_specs=(
            pl.BlockSpec(memory_space=pltpu.MemorySpace.VMEM),
            pl.BlockSpec(memory_space=pltpu.MemorySpace.SMEM),
            pl.BlockSpec(memory_space=pltpu.MemorySpace.SMEM),
        ),
    )(x, fc1_w, fc1_b.reshape(1, H), fc2_w, fc2_b.reshape(1, D),
      fc3_w, fc3_b.reshape(1, H), fc4_w, fc4_b.reshape(1, IN), emb)
    x_reconst, rloss, eloss = out
    rl = rloss[0]
    el = eloss[0]
    return (x_reconst, rl, el, el)


# speedup vs baseline: 1.0025x; 1.0025x over previous
import jax
import jax.numpy as jnp
from jax.experimental import pallas as pl

B = 1024
IN = 784
H = 400
D = 256
K = 512


def _mmb(a, b_t):
    return jax.lax.dot_general(a.astype(jnp.bfloat16),
                               b_t.astype(jnp.bfloat16),
                               (((1,), (1,)), ((), ())),
                               preferred_element_type=jnp.float32)


def _mm_hi(a, b_t):
    return jax.lax.dot_general(a, b_t, (((1,), (1,)), ((), ())),
                               preferred_element_type=jnp.float32,
                               precision=jax.lax.Precision.HIGHEST)


def _vqvae_kernel(x_ref, fc1_w_ref, fc1_b_ref, fc2_w_ref, fc2_b_ref,
                  fc3_w_ref, fc3_b_ref, fc4_w_ref, fc4_b_ref, emb_ref,
                  xr_ref, rloss_ref, eloss_ref):
    f32 = jnp.float32
    x = x_ref[...]
    h1 = jnp.maximum(_mmb(x, fc1_w_ref[...]) + fc1_b_ref[...], 0.0)
    z_e = _mmb(h1, fc2_w_ref[...]) + fc2_b_ref[...]
    emb = emb_ref[...]
    g = _mm_hi(z_e, emb)
    emb_sq = _mm_hi(jnp.ones((1, D), f32), emb * emb)
    score = emb_sq - 2.0 * g
    m = jnp.min(score, axis=1, keepdims=True)
    lane = jax.lax.broadcasted_iota(jnp.int32, (B, K), 1)
    idx = jnp.min(jnp.where(score == m, lane, K), axis=1, keepdims=True)
    onehot = (lane == idx).astype(jnp.bfloat16)
    z_q = jax.lax.dot_general(onehot, emb.astype(jnp.bfloat16),
                              (((1,), (0,)), ((), ())),
                              preferred_element_type=f32)
    h3 = jnp.maximum(_mmb(z_q, fc3_w_ref[...]) + fc3_b_ref[...], 0.0)
    logits = _mmb(h3, fc4_w_ref[...]) + fc4_b_ref[...]
    en = jnp.exp(-logits)
    x_reconst = 1.0 / (1.0 + en)
    xr_ref[...] = x_reconst
    c = jnp.log(1.0 + en)
    logp = jnp.maximum(-c, -100.0)
    log1mp = jnp.maximum(-logits - c, -100.0)
    rloss_ref[0] = -jnp.sum(x * (logp - log1mp) + log1mp) / (B * IN)
    eloss_ref[0] = (jnp.sum(z_e * z_e) + jnp.sum(m)) / B


def kernel(x, fc1_w, fc1_b, fc2_w, fc2_b, fc3_w, fc3_b, fc4_w, fc4_b, emb):
    from jax.experimental.pallas import tpu as pltpu
    out = pl.pallas_call(
        _vqvae_kernel,
        out_shape=(
            jax.ShapeDtypeStruct((B, IN), jnp.float32),
            jax.ShapeDtypeStruct((1,), jnp.float32),
            jax.ShapeDtypeStruct((1,), jnp.float32),
        ),
        out_specs=(
            pl.BlockSpec(memory_space=pltpu.MemorySpace.VMEM),
            pl.BlockSpec(memory_space=pltpu.MemorySpace.SMEM),
            pl.BlockSpec(memory_space=pltpu.MemorySpace.SMEM),
        ),
    )(x, fc1_w, fc1_b.reshape(1, H), fc2_w, fc2_b.reshape(1, D),
      fc3_w, fc3_b.reshape(1, H), fc4_w, fc4_b.reshape(1, IN), emb)
    x_reconst, rloss, eloss = out
    rl = rloss[0]
    el = eloss[0]
    return (x_reconst, rl, el, el)
